# Initial kernel scaffold; baseline (speedup 1.0000x reference)
import jax
import jax.numpy as jnp
from jax.experimental import pallas as pl


def kernel(query, key_value, Wq, bq, Wk, bk, Wv, bv, Wo, bo, ln1_g, ln1_b, ln2_g, ln2_b, gate_up, down):
    raise NotImplementedError("write your pallas kernel here")


# speedup vs baseline: 1.3273x; 1.3273x over previous
import functools

import jax
import jax.numpy as jnp
from jax.experimental import pallas as pl

H = 1024
NH = 16
HD = H // NH
E = 8
I = H * 4 // E
EPS = 1e-06
SCALE = HD ** -0.5


def _ln_rows(x, g, b):
    mu = jnp.mean(x, axis=-1, keepdims=True)
    var = jnp.mean((x - mu) ** 2, axis=-1, keepdims=True)
    return (x - mu) * jax.lax.rsqrt(var + EPS) * g + b


def _qkv_body(x_ref, kv_ref, wq_ref, bq_ref, wk_ref, bk_ref, wv_ref, bv_ref,
              g_ref, b_ref, q_ref, k_ref, v_ref):
    xn = _ln_rows(x_ref[...], g_ref[...], b_ref[...])
    q_ref[...] = jnp.dot(xn, wq_ref[...], preferred_element_type=jnp.float32) + bq_ref[...]
    kv = kv_ref[...]
    k_ref[...] = jnp.dot(kv, wk_ref[...], preferred_element_type=jnp.float32) + bk_ref[...]
    v_ref[...] = jnp.dot(kv, wv_ref[...], preferred_element_type=jnp.float32) + bv_ref[...]


def _attn_body(q_ref, k_ref, v_ref, o_ref):
    q = q_ref[0]
    k = k_ref[0]
    v = v_ref[0]
    outs = []
    for hh in range(2):
        sl = slice(hh * HD, (hh + 1) * HD)
        s = jax.lax.dot_general(q[:, sl], k[:, sl], (((1,), (1,)), ((), ())),
                                preferred_element_type=jnp.float32) * SCALE
        m = jnp.max(s, axis=-1, keepdims=True)
        p = jnp.exp(s - m)
        p = p / jnp.sum(p, axis=-1, keepdims=True)
        outs.append(jnp.dot(p, v[:, sl], preferred_element_type=jnp.float32))
    o_ref[0] = jnp.concatenate(outs, axis=1)


def _oproj_body(o_ref, res_ref, wo_ref, bo_ref, g_ref, b_ref, x_ref, y_ref):
    x = res_ref[...] + jnp.dot(o_ref[...], wo_ref[...],
                               preferred_element_type=jnp.float32) + bo_ref[...]
    x_ref[...] = x
    y_ref[...] = _ln_rows(x, g_ref[...], b_ref[...])


def _moe_body(y_ref, res_ref, gu_ref, dn_ref, out_ref):
    x = y_ref[...]
    gu = jnp.dot(x, gu_ref[0], preferred_element_type=jnp.float32)
    gate = gu[:, :I]
    up = gu[:, I:]
    inter = gate * jax.nn.sigmoid(gate) * up
    out_ref[...] = res_ref[...] + jnp.dot(inter, dn_ref[0],
                                          preferred_element_type=jnp.float32)


def kernel(query, key_value, Wq, bq, Wk, bk, Wv, bv, Wo, bo,
           ln1_g, ln1_b, ln2_g, ln2_b, gate_up, down):
    b, lq, _ = query.shape
    lkv = key_value.shape[1]
    rows = b * lq
    BR = 512
    xf = query.reshape(rows, H)
    kvf = key_value.reshape(b * lkv, H)
    row2 = lambda a: a.reshape(1, H)
    full_w = pl.BlockSpec((H, H), lambda i: (0, 0))
    full_b = pl.BlockSpec((1, H), lambda i: (0, 0))
    rb = pl.BlockSpec((BR, H), lambda i: (i, 0))

    q, k, v = pl.pallas_call(
        _qkv_body,
        grid=(rows // BR,),
        in_specs=[rb, rb, full_w, full_b, full_w, full_b, full_w, full_b,
                  full_b, full_b],
        out_specs=[rb, rb, rb],
        out_shape=[jax.ShapeDtypeStruct((rows, H), jnp.float32)] * 3,
    )(xf, kvf, Wq, row2(bq), Wk, row2(bk), Wv, row2(bv),
      row2(ln1_g), row2(ln1_b))

    q = q.reshape(b, lq, H)
    k = k.reshape(b, lkv, H)
    v = v.reshape(b, lkv, H)

    BQ = 512
    HP = 2 * HD
    o = pl.pallas_call(
        _attn_body,
        grid=(b, NH // 2, lq // BQ),
        in_specs=[
            pl.BlockSpec((1, BQ, HP), lambda bi, h, i: (bi, i, h)),
            pl.BlockSpec((1, lkv, HP), lambda bi, h, i: (bi, 0, h)),
            pl.BlockSpec((1, lkv, HP), lambda bi, h, i: (bi, 0, h)),
        ],
        out_specs=pl.BlockSpec((1, BQ, HP), lambda bi, h, i: (bi, i, h)),
        out_shape=jax.ShapeDtypeStruct((b, lq, H), jnp.float32),
    )(q, k, v)

    of = o.reshape(rows, H)
    x, y = pl.pallas_call(
        _oproj_body,
        grid=(rows // BR,),
        in_specs=[rb, rb, full_w, full_b, full_b, full_b],
        out_specs=[rb, rb],
        out_shape=[jax.ShapeDtypeStruct((rows, H), jnp.float32)] * 2,
    )(of, xf, Wo, row2(bo), row2(ln2_g), row2(ln2_b))

    n = rows // E
    y2 = y.reshape(n, E * H)
    x2 = x.reshape(n, E * H)
    out = pl.pallas_call(
        _moe_body,
        grid=(E,),
        in_specs=[
            pl.BlockSpec((n, H), lambda e: (0, e)),
            pl.BlockSpec((n, H), lambda e: (0, e)),
            pl.BlockSpec((1, H, 2 * I), lambda e: (e, 0, 0)),
            pl.BlockSpec((1, I, H), lambda e: (e, 0, 0)),
        ],
        out_specs=pl.BlockSpec((n, H), lambda e: (0, e)),
        out_shape=jax.ShapeDtypeStruct((n, E * H), jnp.float32),
    )(y2, x2, gate_up, down)
    return out.reshape(b, lq, H)
